# Initial kernel scaffold; baseline (speedup 1.0000x reference)
import jax
import jax.numpy as jnp
from jax.experimental import pallas as pl


def kernel(inputs):
    raise NotImplementedError("write your pallas kernel here")


# speedup vs baseline: 12.6163x; 12.6163x over previous
import jax
import jax.numpy as jnp
import numpy as np
from jax.experimental import pallas as pl
from jax.experimental.pallas import tpu as pltpu

NBINS = 20
NCH = 16
C320 = NCH * NBINS


def _vi_body(nblk, ntot, x_ref, sel_ref, out_ref, mm_ref, j_ref):
    p = pl.program_id(0)
    k = pl.program_id(1)

    @pl.when((p == 0) & (k == 0))
    def _init_mm():
        mm_ref[0:1, :] = jnp.full((1, NCH), jnp.inf, jnp.float32)
        mm_ref[1:2, :] = jnp.full((1, NCH), -jnp.inf, jnp.float32)

    @pl.when(p == 0)
    def _minmax():
        xb = x_ref[...]
        mm_ref[0:1, :] = jnp.minimum(mm_ref[0:1, :], jnp.min(xb, axis=0, keepdims=True))
        mm_ref[1:2, :] = jnp.maximum(mm_ref[1:2, :], jnp.max(xb, axis=0, keepdims=True))

    @pl.when((p == 1) & (k == 0))
    def _init_j():
        j_ref[...] = jnp.zeros_like(j_ref)

    @pl.when(p == 1)
    def _accum():
        xb = x_ref[...]
        lo = mm_ref[0:1, :]
        hi = mm_ref[1:2, :]
        w = (hi - lo) / NBINS
        ix = jnp.clip(((xb - lo) / w).astype(jnp.int32), 0, NBINS - 1)
        ch = jax.lax.broadcasted_iota(jnp.int32, ix.shape, 1)
        comb = (ix + NBINS * ch).astype(jnp.float32)
        combe = jnp.dot(comb, sel_ref[...], preferred_element_type=jnp.float32)
        col = jax.lax.broadcasted_iota(jnp.int32, combe.shape, 1)
        oh = (combe.astype(jnp.int32) == col).astype(jnp.bfloat16)
        j_ref[...] += jax.lax.dot_general(
            oh, oh, (((0,), (0,)), ((), ())), preferred_element_type=jnp.float32)

    @pl.when((p == 1) & (k == nblk - 1))
    def _final():
        J = j_ref[...]
        nf = float(ntot)
        sel = sel_ref[...]
        pc = jnp.sum(J, axis=1, keepdims=True) / NCH / nf
        pr = jnp.sum(J, axis=0, keepdims=True) / NCH / nf
        plogp = pc * jnp.log(pc + 1e-10)
        jp = J / nf
        M = jp * jnp.log(jp / (pc * pr) + 1e-10)
        t1 = jax.lax.dot_general(
            sel, M, (((1,), (0,)), ((), ())), preferred_element_type=jnp.float32)
        mi16 = jax.lax.dot_general(
            t1, sel, (((1,), (1,)), ((), ())), preferred_element_type=jnp.float32)
        hc = -jax.lax.dot_general(
            sel, plogp, (((1,), (0,)), ((), ())), preferred_element_type=jnp.float32)
        hr = -jax.lax.dot_general(
            plogp, sel, (((0,), (1,)), ((), ())), preferred_element_type=jnp.float32)
        vi = hc + hr - 2.0 * mi16
        rr = jax.lax.broadcasted_iota(jnp.int32, (NCH, NCH), 0)
        cc = jax.lax.broadcasted_iota(jnp.int32, (NCH, NCH), 1)
        out_ref[...] = jnp.where(rr == cc, 0.0, vi)


def kernel(inputs):
    B, T, A = inputs.shape
    ntot = B * T
    x = inputs.reshape(ntot, A)
    tblk = 4096 if ntot % 4096 == 0 else ntot
    nblk = ntot // tblk
    sel = jnp.asarray(
        (np.arange(C320)[None, :] // NBINS == np.arange(NCH)[:, None]),
        dtype=jnp.float32)

    import functools
    body = functools.partial(_vi_body, nblk, ntot)
    out16 = pl.pallas_call(
        body,
        grid=(2, nblk),
        in_specs=[
            pl.BlockSpec((tblk, NCH), lambda p, k: (k, 0)),
            pl.BlockSpec((NCH, C320), lambda p, k: (0, 0)),
        ],
        out_specs=pl.BlockSpec((NCH, NCH), lambda p, k: (0, 0)),
        out_shape=jax.ShapeDtypeStruct((NCH, NCH), jnp.float32),
        scratch_shapes=[
            pltpu.VMEM((2, NCH), jnp.float32),
            pltpu.VMEM((C320, C320), jnp.float32),
        ],
        compiler_params=pltpu.CompilerParams(
            dimension_semantics=("arbitrary", "arbitrary")),
    )(x, sel)
    return jnp.broadcast_to(out16[None, :, :], (B, A, A))
